# Initial kernel scaffold; baseline (speedup 1.0000x reference)
import jax
import jax.numpy as jnp
from jax.experimental import pallas as pl


def kernel(tokens, W1, b1, W2, b2):
    raise NotImplementedError("write your pallas kernel here")


# speedup vs baseline: 1.8413x; 1.8413x over previous
import jax
import jax.numpy as jnp
from jax.experimental import pallas as pl
from jax.experimental.pallas import tpu as pltpu

_K = 8
_E = 64
_BM = 256


def _gate_body(x_ref, w1_ref, b1_ref, w2_ref, b2_ref, r_ref, idx_ref):
    x = x_ref[...]
    h = jax.lax.dot_general(
        x, w1_ref[...], (((1,), (1,)), ((), ())),
        preferred_element_type=jnp.float32)
    h = jnp.maximum(h + b1_ref[...], 0.0)
    logits = jax.lax.dot_general(
        h, w2_ref[...], (((1,), (1,)), ((), ())),
        preferred_element_type=jnp.float32)
    logits = logits + b2_ref[...]
    m = jnp.max(logits, axis=-1, keepdims=True)
    e = jnp.exp(logits - m)
    probs = e / jnp.sum(e, axis=-1, keepdims=True)

    work = probs
    iota = jax.lax.broadcasted_iota(jnp.int32, probs.shape, 1)
    cols = []
    for _ in range(_K):
        mx = jnp.max(work, axis=-1, keepdims=True)
        is_max = work == mx
        idx = jnp.min(jnp.where(is_max, iota, _E), axis=-1, keepdims=True)
        cols.append(idx)
        work = jnp.where(iota == idx, -1.0, work)
    idx_ref[...] = jnp.concatenate(cols, axis=1)
    r_ref[...] = jnp.where(work < 0.0, probs, 0.0)


def kernel(tokens, W1, b1, W2, b2):
    B = tokens.shape[0]
    x = tokens.reshape(B, -1)
    D = x.shape[1]
    H = W1.shape[0]
    grid = (B // _BM,)
    r, idx = pl.pallas_call(
        _gate_body,
        grid=grid,
        in_specs=[
            pl.BlockSpec((_BM, D), lambda i: (i, 0)),
            pl.BlockSpec((H, D), lambda i: (0, 0)),
            pl.BlockSpec((1, H), lambda i: (0, 0)),
            pl.BlockSpec((_E, H), lambda i: (0, 0)),
            pl.BlockSpec((1, _E), lambda i: (0, 0)),
        ],
        out_specs=[
            pl.BlockSpec((_BM, _E), lambda i: (i, 0)),
            pl.BlockSpec((_BM, _K), lambda i: (i, 0)),
        ],
        out_shape=[
            jax.ShapeDtypeStruct((B, _E), jnp.float32),
            jax.ShapeDtypeStruct((B, _K), jnp.int32),
        ],
    )(x, W1, b1.reshape(1, H), W2, b2.reshape(1, _E))
    return (r, idx)


# speedup vs baseline: 2.6285x; 1.4275x over previous
import jax
import jax.numpy as jnp
from jax.experimental import pallas as pl
from jax.experimental.pallas import tpu as pltpu

_K = 8
_E = 64
_BM = 256


def _gate_body(x_ref, w1_ref, b1_ref, w2_ref, b2_ref, r_ref, idx_ref):
    h = None
    for j in range(x_ref.shape[1]):
        part = jax.lax.dot_general(
            x_ref[:, j, :], w1_ref[:, j * 1024:(j + 1) * 1024],
            (((1,), (1,)), ((), ())),
            preferred_element_type=jnp.float32)
        h = part if h is None else h + part
    h = jnp.maximum(h + b1_ref[...], 0.0)
    logits = jax.lax.dot_general(
        h, w2_ref[...], (((1,), (1,)), ((), ())),
        preferred_element_type=jnp.float32)
    logits = logits + b2_ref[...]
    m = jnp.max(logits, axis=-1, keepdims=True)
    e = jnp.exp(logits - m)
    probs = e / jnp.sum(e, axis=-1, keepdims=True)

    work = probs
    iota = jax.lax.broadcasted_iota(jnp.int32, probs.shape, 1)
    cols = []
    for _ in range(_K):
        mx = jnp.max(work, axis=-1, keepdims=True)
        is_max = work == mx
        idx = jnp.min(jnp.where(is_max, iota, _E), axis=-1, keepdims=True)
        cols.append(idx)
        work = jnp.where(iota == idx, -1.0, work)
    idx_ref[...] = jnp.concatenate(cols, axis=1)
    r_ref[...] = jnp.where(work < 0.0, probs, 0.0)


def kernel(tokens, W1, b1, W2, b2):
    B, G, Dg = tokens.shape
    D = G * Dg
    H = W1.shape[0]
    grid = (B // _BM,)
    r, idx = pl.pallas_call(
        _gate_body,
        grid=grid,
        in_specs=[
            pl.BlockSpec((_BM, G, Dg), lambda i: (i, 0, 0)),
            pl.BlockSpec((H, D), lambda i: (0, 0)),
            pl.BlockSpec((1, H), lambda i: (0, 0)),
            pl.BlockSpec((_E, H), lambda i: (0, 0)),
            pl.BlockSpec((1, _E), lambda i: (0, 0)),
        ],
        out_specs=[
            pl.BlockSpec((_BM, _E), lambda i: (i, 0)),
            pl.BlockSpec((_BM, _K), lambda i: (i, 0)),
        ],
        out_shape=[
            jax.ShapeDtypeStruct((B, _E), jnp.float32),
            jax.ShapeDtypeStruct((B, _K), jnp.int32),
        ],
    )(tokens, W1, b1.reshape(1, H), W2, b2.reshape(1, _E))
    return (r, idx)


# speedup vs baseline: 3.0221x; 1.1498x over previous
import functools

import jax
import jax.numpy as jnp
from jax import lax
from jax.experimental import pallas as pl
from jax.experimental.pallas import tpu as pltpu
from jax.experimental.pallas import tpu_sc as plsc

_K = 8
_E = 64
_BM = 256
_RW = 256


def _gate_body(x_ref, w1_ref, b1_ref, w2_ref, b2_ref, pt_ref):
    h = None
    for j in range(x_ref.shape[1]):
        part = jax.lax.dot_general(
            x_ref[:, j, :], w1_ref[:, j * 1024:(j + 1) * 1024],
            (((1,), (1,)), ((), ())),
            preferred_element_type=jnp.float32)
        h = part if h is None else h + part
    h = jnp.maximum(h + b1_ref[...], 0.0)
    logits = jax.lax.dot_general(
        w2_ref[...], h, (((1,), (1,)), ((), ())),
        preferred_element_type=jnp.float32)
    logits = logits + b2_ref[...]
    m = jnp.max(logits, axis=0, keepdims=True)
    e = jnp.exp(logits - m)
    pt_ref[...] = e / jnp.sum(e, axis=0, keepdims=True)


def _topk_body(pt_hbm, r_hbm, idx_hbm, pv, rv, iv):
    wid = lax.axis_index("s") * 2 + lax.axis_index("c")
    base = wid * _RW
    pltpu.sync_copy(pt_hbm.at[:, pl.ds(base, _RW)], pv)

    def _zero(i, _):
        rv[pl.ds(i * 16, 16)] = jnp.zeros((16,), jnp.float32)
        return _
    lax.fori_loop(0, _RW * _E // 16, _zero, 0)

    lane = lax.iota(jnp.int32, 16)
    for g in range(_RW // 16):
        rows = g * 16 + lane

        def _insert(e, carry):
            t = list(carry[:_K])
            ti = list(carry[_K:])
            v = pv[e, pl.ds(g * 16, 16)]
            vi = jnp.full((16,), 0, jnp.int32) + e
            for j in range(_K):
                c = v > t[j]
                t[j], v = jnp.where(c, v, t[j]), jnp.where(c, t[j], v)
                ti[j], vi = jnp.where(c, vi, ti[j]), jnp.where(c, ti[j], vi)
            return tuple(t) + tuple(ti)

        init = tuple(jnp.full((16,), -1.0, jnp.float32) for _ in range(_K)) \
            + tuple(jnp.zeros((16,), jnp.int32) for _ in range(_K))
        res = lax.fori_loop(0, _E, _insert, init)
        for j in range(_K):
            plsc.store_scatter(iv, [rows * _K + j], res[_K + j])
            plsc.store_scatter(rv, [rows * _E + res[_K + j]], res[j])

    pltpu.sync_copy(rv, r_hbm.at[pl.ds(base * _E, _RW * _E)])
    pltpu.sync_copy(iv, idx_hbm.at[pl.ds(base * _K, _RW * _K)])


def kernel(tokens, W1, b1, W2, b2):
    B, G, Dg = tokens.shape
    D = G * Dg
    H = W1.shape[0]
    grid = (B // _BM,)
    probsT = pl.pallas_call(
        _gate_body,
        grid=grid,
        in_specs=[
            pl.BlockSpec((_BM, G, Dg), lambda i: (i, 0, 0)),
            pl.BlockSpec((H, D), lambda i: (0, 0)),
            pl.BlockSpec((1, H), lambda i: (0, 0)),
            pl.BlockSpec((_E, H), lambda i: (0, 0)),
            pl.BlockSpec((_E, 1), lambda i: (0, 0)),
        ],
        out_specs=pl.BlockSpec((_E, _BM), lambda i: (0, i)),
        out_shape=jax.ShapeDtypeStruct((_E, B), jnp.float32),
    )(tokens, W1, b1.reshape(1, H), W2, b2.reshape(_E, 1))

    mesh = plsc.VectorSubcoreMesh(core_axis_name="c", subcore_axis_name="s")
    r, idx = pl.kernel(
        _topk_body,
        mesh=mesh,
        compiler_params=pltpu.CompilerParams(needs_layout_passes=False),
        out_type=[
            jax.ShapeDtypeStruct((B * _E,), jnp.float32),
            jax.ShapeDtypeStruct((B * _K,), jnp.int32),
        ],
        scratch_types=[
            pltpu.VMEM((_E, _RW), jnp.float32),
            pltpu.VMEM((_RW * _E,), jnp.float32),
            pltpu.VMEM((_RW * _K,), jnp.int32),
        ],
    )(probsT)
    return (r.reshape(B, _E), idx.reshape(B, _K))


# speedup vs baseline: 3.0914x; 1.0229x over previous
import functools

import jax
import jax.numpy as jnp
from jax import lax
from jax.experimental import pallas as pl
from jax.experimental.pallas import tpu as pltpu
from jax.experimental.pallas import tpu_sc as plsc

_K = 8
_E = 64
_BM = 256
_RW = 256


def _gate_body(x_ref, w1_ref, b1_ref, w2_ref, b2_ref, pt_ref):
    h = None
    for j in range(x_ref.shape[1]):
        part = jax.lax.dot_general(
            x_ref[:, j, :], w1_ref[:, j * 1024:(j + 1) * 1024],
            (((1,), (1,)), ((), ())),
            preferred_element_type=jnp.float32)
        h = part if h is None else h + part
    h = jnp.maximum(h + b1_ref[...], 0.0)
    logits = jax.lax.dot_general(
        w2_ref[...], h, (((1,), (1,)), ((), ())),
        preferred_element_type=jnp.float32)
    logits = logits + b2_ref[...]
    m = jnp.max(logits, axis=0, keepdims=True)
    e = jnp.exp(logits - m)
    pt_ref[...] = e / jnp.sum(e, axis=0, keepdims=True)


def _topk_body(pt_hbm, r_hbm, idx_hbm, pv, rv, iv):
    wid = lax.axis_index("s") * 2 + lax.axis_index("c")
    base = wid * _RW
    pltpu.sync_copy(pt_hbm.at[:, pl.ds(base, _RW)], pv)

    def _zero(r, _):
        for c in range(_E // 16):
            rv[r, pl.ds(c * 16, 16)] = jnp.zeros((16,), jnp.float32)
        return _
    lax.fori_loop(0, _RW, _zero, 0)

    lane = lax.iota(jnp.int32, 16)
    for g in range(_RW // 16):
        rows = g * 16 + lane

        def _insert(e, carry):
            t = list(carry[:_K])
            ti = list(carry[_K:])
            v = pv[e, pl.ds(g * 16, 16)]
            vi = jnp.full((16,), 0, jnp.int32) + e
            for j in range(_K):
                c = v > t[j]
                t[j], v = jnp.where(c, v, t[j]), jnp.where(c, t[j], v)
                ti[j], vi = jnp.where(c, vi, ti[j]), jnp.where(c, ti[j], vi)
            return tuple(t) + tuple(ti)

        init = tuple(jnp.full((16,), -1.0, jnp.float32) for _ in range(_K)) \
            + tuple(jnp.zeros((16,), jnp.int32) for _ in range(_K))
        res = lax.fori_loop(0, _E, _insert, init)
        for j in range(_K):
            plsc.store_scatter(iv, [rows, jnp.full((16,), j, jnp.int32)],
                               res[_K + j])
            plsc.store_scatter(rv, [rows, res[_K + j]], res[j])

    pltpu.sync_copy(rv, r_hbm.at[pl.ds(base, _RW), :])
    pltpu.sync_copy(iv, idx_hbm.at[pl.ds(base, _RW), :])


def kernel(tokens, W1, b1, W2, b2):
    B, G, Dg = tokens.shape
    D = G * Dg
    H = W1.shape[0]
    grid = (B // _BM,)
    probsT = pl.pallas_call(
        _gate_body,
        grid=grid,
        in_specs=[
            pl.BlockSpec((_BM, G, Dg), lambda i: (i, 0, 0)),
            pl.BlockSpec((H, D), lambda i: (0, 0)),
            pl.BlockSpec((1, H), lambda i: (0, 0)),
            pl.BlockSpec((_E, H), lambda i: (0, 0)),
            pl.BlockSpec((_E, 1), lambda i: (0, 0)),
        ],
        out_specs=pl.BlockSpec((_E, _BM), lambda i: (0, i)),
        out_shape=jax.ShapeDtypeStruct((_E, B), jnp.float32),
    )(tokens, W1, b1.reshape(1, H), W2, b2.reshape(_E, 1))

    mesh = plsc.VectorSubcoreMesh(core_axis_name="c", subcore_axis_name="s")
    r, idx = pl.kernel(
        _topk_body,
        mesh=mesh,
        compiler_params=pltpu.CompilerParams(needs_layout_passes=False),
        out_type=[
            jax.ShapeDtypeStruct((B, _E), jnp.float32),
            jax.ShapeDtypeStruct((B, _K), jnp.int32),
        ],
        scratch_types=[
            pltpu.VMEM((_E, _RW), jnp.float32),
            pltpu.VMEM((_RW, _E), jnp.float32),
            pltpu.VMEM((_RW, _K), jnp.int32),
        ],
    )(probsT)
    return (r, idx)


# speedup vs baseline: 3.2622x; 1.0553x over previous
import functools

import jax
import jax.numpy as jnp
from jax import lax
from jax.experimental import pallas as pl
from jax.experimental.pallas import tpu as pltpu
from jax.experimental.pallas import tpu_sc as plsc

_K = 8
_E = 64
_BM = 256
_RW = 256


def _gate_body(x_ref, w1_ref, b1_ref, w2_ref, b2_ref, pt_ref):
    ht = None
    for j in range(x_ref.shape[1]):
        part = jax.lax.dot_general(
            w1_ref[:, j * 1024:(j + 1) * 1024], x_ref[:, j, :],
            (((1,), (1,)), ((), ())),
            preferred_element_type=jnp.float32)
        ht = part if ht is None else ht + part
    ht = jnp.maximum(ht + b1_ref[...], 0.0)
    logits = jax.lax.dot_general(
        w2_ref[...], ht, (((1,), (0,)), ((), ())),
        preferred_element_type=jnp.float32)
    logits = logits + b2_ref[...]
    m = jnp.max(logits, axis=0, keepdims=True)
    e = jnp.exp(logits - m)
    pt_ref[...] = e / jnp.sum(e, axis=0, keepdims=True)


def _topk_body(pt_hbm, r_hbm, idx_hbm, pv, rv, iv):
    wid = lax.axis_index("s") * 2 + lax.axis_index("c")
    base = wid * _RW
    pltpu.sync_copy(pt_hbm.at[:, pl.ds(base, _RW)], pv)

    def _zero(r, _):
        for c in range(_E // 16):
            rv[r, pl.ds(c * 16, 16)] = jnp.zeros((16,), jnp.float32)
        return _
    lax.fori_loop(0, _RW, _zero, 0)

    lane = lax.iota(jnp.int32, 16)
    for g in range(_RW // 16):
        rows = g * 16 + lane

        def _insert(e, carry):
            t = list(carry[:_K])
            ti = list(carry[_K:])
            v = pv[e, pl.ds(g * 16, 16)]
            vi = jnp.full((16,), 0, jnp.int32) + e
            for j in range(_K):
                c = v > t[j]
                t[j], v = jnp.where(c, v, t[j]), jnp.where(c, t[j], v)
                ti[j], vi = jnp.where(c, vi, ti[j]), jnp.where(c, ti[j], vi)
            return tuple(t) + tuple(ti)

        init = tuple(jnp.full((16,), -1.0, jnp.float32) for _ in range(_K)) \
            + tuple(jnp.zeros((16,), jnp.int32) for _ in range(_K))
        res = lax.fori_loop(0, _E, _insert, init)
        for j in range(_K):
            plsc.store_scatter(iv, [rows, jnp.full((16,), j, jnp.int32)],
                               res[_K + j])
            plsc.store_scatter(rv, [rows, res[_K + j]], res[j])

    pltpu.sync_copy(rv, r_hbm.at[pl.ds(base, _RW), :])
    pltpu.sync_copy(iv, idx_hbm.at[pl.ds(base, _RW), :])


def kernel(tokens, W1, b1, W2, b2):
    B, G, Dg = tokens.shape
    D = G * Dg
    H = W1.shape[0]
    grid = (B // _BM,)
    probsT = pl.pallas_call(
        _gate_body,
        grid=grid,
        in_specs=[
            pl.BlockSpec((_BM, G, Dg), lambda i: (i, 0, 0)),
            pl.BlockSpec((H, D), lambda i: (0, 0)),
            pl.BlockSpec((H, 1), lambda i: (0, 0)),
            pl.BlockSpec((_E, H), lambda i: (0, 0)),
            pl.BlockSpec((_E, 1), lambda i: (0, 0)),
        ],
        out_specs=pl.BlockSpec((_E, _BM), lambda i: (0, i)),
        out_shape=jax.ShapeDtypeStruct((_E, B), jnp.float32),
    )(tokens, W1, b1.reshape(H, 1), W2, b2.reshape(_E, 1))

    mesh = plsc.VectorSubcoreMesh(core_axis_name="c", subcore_axis_name="s")
    r, idx = pl.kernel(
        _topk_body,
        mesh=mesh,
        compiler_params=pltpu.CompilerParams(needs_layout_passes=False),
        out_type=[
            jax.ShapeDtypeStruct((B, _E), jnp.float32),
            jax.ShapeDtypeStruct((B, _K), jnp.int32),
        ],
        scratch_types=[
            pltpu.VMEM((_E, _RW), jnp.float32),
            pltpu.VMEM((_RW, _E), jnp.float32),
            pltpu.VMEM((_RW, _K), jnp.int32),
        ],
    )(probsT)
    return (r, idx)


# speedup vs baseline: 3.5331x; 1.0831x over previous
import functools

import jax
import jax.numpy as jnp
from jax import lax
from jax.experimental import pallas as pl
from jax.experimental.pallas import tpu as pltpu
from jax.experimental.pallas import tpu_sc as plsc

_K = 8
_E = 64
_BM = 512
_RW = 256


def _gate_body(x_ref, w1_ref, b1_ref, w2_ref, b2_ref, pt_ref):
    ht = None
    for j in range(x_ref.shape[1]):
        part = jax.lax.dot_general(
            w1_ref[:, j * 1024:(j + 1) * 1024], x_ref[:, j, :],
            (((1,), (1,)), ((), ())),
            preferred_element_type=jnp.float32)
        ht = part if ht is None else ht + part
    ht = jnp.maximum(ht + b1_ref[...], 0.0)
    logits = jax.lax.dot_general(
        w2_ref[...], ht, (((1,), (0,)), ((), ())),
        preferred_element_type=jnp.float32)
    logits = logits + b2_ref[...]
    m = jnp.max(logits, axis=0, keepdims=True)
    e = jnp.exp(logits - m)
    pt_ref[...] = e / jnp.sum(e, axis=0, keepdims=True)


def _topk_body(pt_hbm, r_hbm, idx_hbm, pv, rv, iv):
    wid = lax.axis_index("s") * 2 + lax.axis_index("c")
    base = wid * _RW
    pltpu.sync_copy(pt_hbm.at[:, pl.ds(base, _RW)], pv)

    def _zero(r, _):
        for c in range(_E // 16):
            rv[r, pl.ds(c * 16, 16)] = jnp.zeros((16,), jnp.float32)
        return _
    lax.fori_loop(0, _RW, _zero, 0)

    lane = lax.iota(jnp.int32, 16)
    for g in range(_RW // 16):
        rows = g * 16 + lane

        def _insert(e, carry):
            t = list(carry[:_K])
            ti = list(carry[_K:])
            v = pv[e, pl.ds(g * 16, 16)]
            vi = jnp.full((16,), 0, jnp.int32) + e
            for j in range(_K):
                c = v > t[j]
                t[j], v = jnp.where(c, v, t[j]), jnp.where(c, t[j], v)
                ti[j], vi = jnp.where(c, vi, ti[j]), jnp.where(c, ti[j], vi)
            return tuple(t) + tuple(ti)

        init = tuple(jnp.full((16,), -1.0, jnp.float32) for _ in range(_K)) \
            + tuple(jnp.zeros((16,), jnp.int32) for _ in range(_K))
        res = lax.fori_loop(0, _E, _insert, init)
        for j in range(_K):
            plsc.store_scatter(iv, [rows, jnp.full((16,), j, jnp.int32)],
                               res[_K + j])
            plsc.store_scatter(rv, [rows, res[_K + j]], res[j])

    pltpu.sync_copy(rv, r_hbm.at[pl.ds(base, _RW), :])
    pltpu.sync_copy(iv, idx_hbm.at[pl.ds(base, _RW), :])


def kernel(tokens, W1, b1, W2, b2):
    B, G, Dg = tokens.shape
    D = G * Dg
    H = W1.shape[0]
    grid = (B // _BM,)
    probsT = pl.pallas_call(
        _gate_body,
        grid=grid,
        in_specs=[
            pl.BlockSpec((_BM, G, Dg), lambda i: (i, 0, 0)),
            pl.BlockSpec((H, D), lambda i: (0, 0)),
            pl.BlockSpec((H, 1), lambda i: (0, 0)),
            pl.BlockSpec((_E, H), lambda i: (0, 0)),
            pl.BlockSpec((_E, 1), lambda i: (0, 0)),
        ],
        out_specs=pl.BlockSpec((_E, _BM), lambda i: (0, i)),
        out_shape=jax.ShapeDtypeStruct((_E, B), jnp.float32),
    )(tokens, W1, b1.reshape(H, 1), W2, b2.reshape(_E, 1))

    mesh = plsc.VectorSubcoreMesh(core_axis_name="c", subcore_axis_name="s")
    r, idx = pl.kernel(
        _topk_body,
        mesh=mesh,
        compiler_params=pltpu.CompilerParams(needs_layout_passes=False),
        out_type=[
            jax.ShapeDtypeStruct((B, _E), jnp.float32),
            jax.ShapeDtypeStruct((B, _K), jnp.int32),
        ],
        scratch_types=[
            pltpu.VMEM((_E, _RW), jnp.float32),
            pltpu.VMEM((_RW, _E), jnp.float32),
            pltpu.VMEM((_RW, _K), jnp.int32),
        ],
    )(probsT)
    return (r, idx)
